# baseline (speedup 1.0000x reference)
import functools

import jax
import jax.numpy as jnp
from jax import lax
from jax.experimental import pallas as pl
from jax.experimental.pallas import tpu as pltpu
from jax.experimental.pallas import tpu_sc as plsc


def _gather_pe(t, pe_weight):
    (n,) = t.shape
    period, c = pe_weight.shape
    info = plsc.get_sparse_core_info()
    nw = info.num_cores * info.num_subcores
    n_per_w = n // nw
    mesh = plsc.VectorSubcoreMesh(core_axis_name="c", subcore_axis_name="s")

    @functools.partial(
        pl.kernel,
        mesh=mesh,
        out_type=jax.ShapeDtypeStruct((n, c), jnp.float32),
        scratch_types=[
            pltpu.VMEM((n_per_w,), jnp.int32),
            pltpu.VMEM((n_per_w, c), jnp.float32),
            pltpu.SemaphoreType.DMA,
        ],
        compiler_params=pltpu.CompilerParams(use_tc_tiling_on_sc=False),
    )
    def gather_kernel(t_hbm, table_hbm, out_hbm, idx_v, rows_v, sem):
        wid = lax.axis_index("s") * info.num_cores + lax.axis_index("c")
        base = wid * n_per_w
        pltpu.sync_copy(t_hbm.at[pl.ds(base, n_per_w)], idx_v)
        for i in range(n_per_w // 16):
            sl = pl.ds(i * 16, 16)
            idx_v[sl] = lax.rem(idx_v[sl], jnp.int32(period))
        pltpu.async_copy(table_hbm.at[idx_v], rows_v, sem).wait()
        pltpu.sync_copy(rows_v, out_hbm.at[pl.ds(base, n_per_w)])

    return gather_kernel(t, pe_weight)


def _broadcast_add_t(img_t, pe_t):
    b, c, n = img_t.shape
    bb = 8

    def body(img_ref, pe_ref, out_ref):
        out_ref[...] = img_ref[...] + pe_ref[...][None]

    return pl.pallas_call(
        body,
        grid=(b // bb,),
        in_specs=[
            pl.BlockSpec((bb, c, n), lambda i: (i, 0, 0)),
            pl.BlockSpec((c, n), lambda i: (0, 0)),
        ],
        out_specs=pl.BlockSpec((bb, c, n), lambda i: (i, 0, 0)),
        out_shape=jax.ShapeDtypeStruct((b, c, n), jnp.float32),
    )(img_t, pe_t)


def _broadcast_add_t_ring(img_t, pe_t):
    b, c, n = img_t.shape
    bb, r_bufs = 8, 3
    num_chunks = b // bb

    def body(img_hbm, pe_ref, out_hbm, in_buf, out_buf, sem_in, sem_out):
        def in_copy(k, r):
            return pltpu.make_async_copy(
                img_hbm.at[pl.ds(k * bb, bb)], in_buf.at[r], sem_in.at[r])

        def out_copy(k, r):
            return pltpu.make_async_copy(
                out_buf.at[r], out_hbm.at[pl.ds(k * bb, bb)], sem_out.at[r])

        for r in range(r_bufs):
            in_copy(r, r).start()

        def step(k, carry):
            r = lax.rem(k, r_bufs)
            in_copy(k, r).wait()

            @pl.when(k >= r_bufs)
            def _():
                out_copy(k - r_bufs, r).wait()

            out_buf[r] = in_buf[r] + pe_ref[...][None]
            out_copy(k, r).start()

            @pl.when(k + r_bufs < num_chunks)
            def _():
                in_copy(k + r_bufs, r).start()

            return carry

        lax.fori_loop(0, num_chunks, step, 0)

        for k in range(num_chunks - r_bufs, num_chunks):
            out_copy(k, k % r_bufs).wait()

    return pl.pallas_call(
        body,
        in_specs=[
            pl.BlockSpec(memory_space=pl.ANY),
            pl.BlockSpec(memory_space=pltpu.VMEM),
        ],
        out_specs=pl.BlockSpec(memory_space=pl.ANY),
        out_shape=jax.ShapeDtypeStruct((b, c, n), jnp.float32),
        scratch_shapes=[
            pltpu.VMEM((r_bufs, bb, c, n), jnp.float32),
            pltpu.VMEM((r_bufs, bb, c, n), jnp.float32),
            pltpu.SemaphoreType.DMA((r_bufs,)),
            pltpu.SemaphoreType.DMA((r_bufs,)),
        ],
    )(img_t, pe_t)


def kernel(image_embeds, t, pe_weight):
    b, n, c = image_embeds.shape
    pe = _gather_pe(t.astype(jnp.int32), pe_weight.astype(jnp.float32))
    img_t = jnp.transpose(image_embeds, (0, 2, 1))
    out_t = _broadcast_add_t_ring(img_t, pe.T)
    return jnp.transpose(out_t, (0, 2, 1))


# speedup vs baseline: 1.0066x; 1.0066x over previous
import functools

import jax
import jax.numpy as jnp
from jax import lax
from jax.experimental import pallas as pl
from jax.experimental.pallas import tpu as pltpu
from jax.experimental.pallas import tpu_sc as plsc


def _gather_pe(t, pe_weight):
    (n,) = t.shape
    period, c = pe_weight.shape
    info = plsc.get_sparse_core_info()
    nw = info.num_cores * info.num_subcores
    n_per_w = n // nw
    mesh = plsc.VectorSubcoreMesh(core_axis_name="c", subcore_axis_name="s")

    @functools.partial(
        pl.kernel,
        mesh=mesh,
        out_type=jax.ShapeDtypeStruct((n, c), jnp.float32),
        scratch_types=[
            pltpu.VMEM((n_per_w,), jnp.int32),
            pltpu.VMEM((n_per_w, c), jnp.float32),
            pltpu.SemaphoreType.DMA,
        ],
        compiler_params=pltpu.CompilerParams(use_tc_tiling_on_sc=False),
    )
    def gather_kernel(t_hbm, table_hbm, out_hbm, idx_v, rows_v, sem):
        wid = lax.axis_index("s") * info.num_cores + lax.axis_index("c")
        base = wid * n_per_w
        pltpu.sync_copy(t_hbm.at[pl.ds(base, n_per_w)], idx_v)
        for i in range(n_per_w // 16):
            sl = pl.ds(i * 16, 16)
            idx_v[sl] = lax.rem(idx_v[sl], jnp.int32(period))
        pltpu.async_copy(table_hbm.at[idx_v], rows_v, sem).wait()
        pltpu.sync_copy(rows_v, out_hbm.at[pl.ds(base, n_per_w)])

    return gather_kernel(t, pe_weight)


def _broadcast_add_t(img_t, pe_t):
    b, c, n = img_t.shape
    bb = 8

    def body(img_ref, pe_ref, out_ref):
        out_ref[...] = img_ref[...] + pe_ref[...][None]

    return pl.pallas_call(
        body,
        grid=(b // bb,),
        in_specs=[
            pl.BlockSpec((bb, c, n), lambda i: (i, 0, 0)),
            pl.BlockSpec((c, n), lambda i: (0, 0)),
        ],
        out_specs=pl.BlockSpec((bb, c, n), lambda i: (i, 0, 0)),
        out_shape=jax.ShapeDtypeStruct((b, c, n), jnp.float32),
    )(img_t, pe_t)


def kernel(image_embeds, t, pe_weight):
    b, n, c = image_embeds.shape
    pe = _gather_pe(t.astype(jnp.int32), pe_weight.astype(jnp.float32))
    img_t = jnp.transpose(image_embeds, (0, 2, 1))
    out_t = _broadcast_add_t(img_t, pe.T)
    return jnp.transpose(out_t, (0, 2, 1))
